# baseline (speedup 1.0000x reference)
import functools

import jax
import jax.numpy as jnp
from jax import lax
from jax.experimental import pallas as pl
from jax.experimental.pallas import tpu as pltpu
from jax.experimental.pallas import tpu_sc as plsc

_NC, _NS, _L = 2, 16, 16
_NW = _NC * _NS


def kernel(user, item, user_table, item_table):
    batch = user.shape[0]
    dim = user_table.shape[1]
    assert batch % (_NW * _L) == 0 and dim == 2 * _L
    bpw = batch // _NW

    mesh = plsc.VectorSubcoreMesh(
        core_axis_name="c", subcore_axis_name="s",
        num_cores=_NC, num_subcores=_NS,
    )

    cp = pltpu.CompilerParams(
        needs_layout_passes=False, use_tc_tiling_on_sc=False)

    @functools.partial(
        pl.kernel,
        out_type=jax.ShapeDtypeStruct((batch,), jnp.float32),
        mesh=mesh,
        compiler_params=cp,
        scratch_types=[
            pltpu.VMEM((bpw,), jnp.int32),
            pltpu.VMEM((bpw,), jnp.int32),
            pltpu.VMEM((bpw, dim), jnp.float32),
            pltpu.VMEM((bpw, dim), jnp.float32),
            pltpu.VMEM((bpw,), jnp.float32),
            pltpu.SemaphoreType.DMA,
            pltpu.SemaphoreType.DMA,
        ],
    )
    def sc_kernel(user_hbm, item_hbm, utab_hbm, itab_hbm, out_hbm,
                  uidx_v, iidx_v, urows_v, irows_v, out_v, sem_u, sem_i):
        wid = lax.axis_index("s") * _NC + lax.axis_index("c")
        base = wid * bpw
        pltpu.sync_copy(user_hbm.at[pl.ds(base, bpw)], uidx_v)
        pltpu.sync_copy(item_hbm.at[pl.ds(base, bpw)], iidx_v)
        cp_u = pltpu.async_copy(utab_hbm.at[uidx_v], urows_v, sem_u)
        cp_i = pltpu.async_copy(itab_hbm.at[iidx_v], irows_v, sem_i)
        cp_u.wait()
        cp_i.wait()

        last_lane = lax.iota(jnp.int32, _L) == (_L - 1)

        @pl.loop(0, bpw, step=_L)
        def _(r0):
            for r in range(_L):
                row = r0 + r
                u0 = urows_v[row, pl.ds(0, _L)]
                u1 = urows_v[row, pl.ds(_L, _L)]
                v0 = irows_v[row, pl.ds(0, _L)]
                v1 = irows_v[row, pl.ds(_L, _L)]
                s = u0 * v0 + u1 * v1
                tot = plsc.cumsum(s)
                idx = jnp.full((_L,), row, jnp.int32)
                plsc.store_scatter(out_v, [idx], tot, mask=last_lane)

        pltpu.sync_copy(out_v, out_hbm.at[pl.ds(base, bpw)])

    return sc_kernel(user, item, user_table, item_table)
